# baseline (speedup 1.0000x reference)
import functools

import jax
import jax.numpy as jnp
from jax import lax
from jax.experimental import pallas as pl
from jax.experimental.pallas import tpu as pltpu
from jax.experimental.pallas import tpu_sc as plsc

N = 10000
E = 320000
D = 128
K = 10
ALPHA = 0.1

NC = 2
NS = 16
NW = NC * NS
U = 128
SUK = 4
NSU = E // (SUK * U)
ZR = 632
AGG_ROWS = NS * ZR
RCH = 64
ZOFFS = (0, 64, 128, 192, 256, 320, 384, 448, 512, 568)

_mesh = plsc.VectorSubcoreMesh(core_axis_name="c", subcore_axis_name="s")


def _tile_ranges(c, s):
    w = c * NS + s
    u0 = (w * NSU) // NW
    u1 = ((w + 1) * NSU) // NW
    return u0, u1 - u0


def _out_rowbase(s):
    return jnp.minimum(s * ZR, N - ZR)


@functools.partial(
    pl.kernel,
    out_type=jax.ShapeDtypeStruct((NC, N, 16), jnp.float32),
    mesh=_mesh,
    scratch_types=[
        pltpu.VMEM((1, 2 * SUK, U), jnp.int32),
        pltpu.VMEM((U, 16), jnp.float32),
        pltpu.VMEM((RCH, 16), jnp.float32),
        pltpu.VMEM_SHARED((AGG_ROWS, 16), jnp.float32),
    ],
)
def _deg_kernel(idx_hbm, deg_hbm, idx_v, ones_v, zero_v, deg_sh):
    c = lax.axis_index("c")
    s = lax.axis_index("s")
    u0, nu = _tile_ranges(c, s)

    def fill(r, _):
        ones_v[r, :] = jnp.full((16,), 1.0, jnp.float32)
        return 0

    lax.fori_loop(0, U, fill, 0)

    def zfill(r, _):
        zero_v[r, :] = jnp.zeros((16,), jnp.float32)
        return 0

    lax.fori_loop(0, RCH, zfill, 0)
    for off in ZOFFS:
        pltpu.sync_copy(zero_v, deg_sh.at[pl.ds(s * ZR + off, RCH)])
    plsc.subcore_barrier()

    def edge_su(i, _):
        pltpu.sync_copy(idx_hbm.at[pl.ds(u0 + i, 1)], idx_v)
        for k in range(SUK):
            pltpu.sync_copy(ones_v, deg_sh.at[idx_v.at[0, SUK + k]],
                            add=True)
        return 0

    lax.fori_loop(0, nu, edge_su, 0)
    plsc.subcore_barrier()

    lr = _out_rowbase(s)
    pltpu.sync_copy(deg_sh.at[pl.ds(lr, ZR)], deg_hbm.at[c, pl.ds(lr, ZR)])


@functools.partial(
    pl.kernel,
    out_type=jax.ShapeDtypeStruct((NC, N, D), jnp.float32),
    mesh=_mesh,
    scratch_types=[
        pltpu.VMEM((1, 2 * SUK, U), jnp.int32),
        pltpu.VMEM((1, 2 * SUK, U), jnp.int32),
        pltpu.VMEM((U, D), jnp.float32),
        pltpu.VMEM((U, D), jnp.float32),
        pltpu.VMEM((RCH, D), jnp.float32),
        pltpu.VMEM_SHARED((AGG_ROWS, D), jnp.float32),
        pltpu.SemaphoreType.DMA,
        pltpu.SemaphoreType.DMA,
        pltpu.SemaphoreType.DMA,
        pltpu.SemaphoreType.DMA,
        pltpu.SemaphoreType.DMA,
    ],
)
def _scatter_kernel(g_hbm, idx_hbm, pout_hbm, idxa_v, idxb_v, gbufa_v,
                    gbufb_v, zero_v, agg_sh, sga, sgb, sia, sib, sz):
    c = lax.axis_index("c")
    s = lax.axis_index("s")
    u0, nu = _tile_ranges(c, s)

    def zfill(r, _):
        for k in range(D // 16):
            zero_v[r, pl.ds(k * 16, 16)] = jnp.zeros((16,), jnp.float32)
        return 0

    lax.fori_loop(0, RCH, zfill, 0)
    for off in ZOFFS:
        pltpu.sync_copy(zero_v, agg_sh.at[pl.ds(s * ZR + off, RCH)])
    plsc.subcore_barrier()

    def process_su(idx_v):
        pending = pltpu.async_copy(g_hbm.at[idx_v.at[0, 0]], gbufa_v, sga)
        for k in range(SUK):
            cur_buf = gbufa_v if k % 2 == 0 else gbufb_v
            nxt = None
            if k < SUK - 1:
                nxt = pltpu.async_copy(
                    g_hbm.at[idx_v.at[0, k + 1]],
                    gbufb_v if k % 2 == 0 else gbufa_v,
                    sgb if k % 2 == 0 else sga)
            pending.wait()
            pltpu.sync_copy(cur_buf, agg_sh.at[idx_v.at[0, SUK + k]],
                            add=True)
            pending = nxt

    def edge_su(i, _):
        pltpu.sync_copy(idx_hbm.at[pl.ds(u0 + i, 1)], idxa_v)
        process_su(idxa_v)
        return 0

    lax.fori_loop(0, nu, edge_su, 0)
    plsc.subcore_barrier()

    lr = _out_rowbase(s)
    pltpu.sync_copy(agg_sh.at[pl.ds(lr, ZR)], pout_hbm.at[c, pl.ds(lr, ZR)])


BLK = 1000


def _mlp_body(x_ref, deg_ref, w1_ref, b1_ref, w2_ref, b2_ref,
              g0_ref, a_ref, c_ref, dinv_ref):
    x = x_ref[...]
    h = jnp.dot(x, w1_ref[...].T, preferred_element_type=jnp.float32)
    h = jnp.maximum(h + b1_ref[...], 0.0)
    h = jnp.dot(h, w2_ref[...].T, preferred_element_type=jnp.float32)
    h = h + b2_ref[...]
    degs = deg_ref[...]
    deg = degs[0, :, 0:1] + degs[1, :, 0:1] + 1.0
    dinv = lax.rsqrt(deg)
    g0 = h * dinv
    g0_ref[...] = g0
    a_ref[...] = jnp.broadcast_to((1.0 - ALPHA) * dinv * dinv, (BLK, 16))
    c_ref[...] = ALPHA * g0
    dinv_ref[...] = dinv


def _mlp_stage(x, deg2, W1, b1, W2, b2):
    grid = (N // BLK,)
    return pl.pallas_call(
        _mlp_body,
        grid=grid,
        in_specs=[
            pl.BlockSpec((BLK, D), lambda i: (i, 0)),
            pl.BlockSpec((NC, BLK, 16), lambda i: (0, i, 0)),
            pl.BlockSpec((D, D), lambda i: (0, 0)),
            pl.BlockSpec((1, D), lambda i: (0, 0)),
            pl.BlockSpec((D, D), lambda i: (0, 0)),
            pl.BlockSpec((1, D), lambda i: (0, 0)),
        ],
        out_specs=[
            pl.BlockSpec((BLK, D), lambda i: (i, 0)),
            pl.BlockSpec((BLK, 16), lambda i: (i, 0)),
            pl.BlockSpec((BLK, D), lambda i: (i, 0)),
            pl.BlockSpec((BLK, 1), lambda i: (i, 0)),
        ],
        out_shape=[
            jax.ShapeDtypeStruct((N, D), jnp.float32),
            jax.ShapeDtypeStruct((N, 16), jnp.float32),
            jax.ShapeDtypeStruct((N, D), jnp.float32),
            jax.ShapeDtypeStruct((N, 1), jnp.float32),
        ],
    )(x, deg2, W1, b1.reshape(1, D), W2, b2.reshape(1, D))


def _combine_body(p_ref, g_ref, a_ref, c_ref, o_ref):
    p = p_ref[...]
    s = p[0] + p[1] + g_ref[...]
    o_ref[...] = a_ref[...][:, 0:1] * s + c_ref[...]


def _combine_stage(pout, g, avec, cvec):
    grid = (N // BLK,)
    return pl.pallas_call(
        _combine_body,
        grid=grid,
        in_specs=[
            pl.BlockSpec((NC, BLK, D), lambda i: (0, i, 0)),
            pl.BlockSpec((BLK, D), lambda i: (i, 0)),
            pl.BlockSpec((BLK, 16), lambda i: (i, 0)),
            pl.BlockSpec((BLK, D), lambda i: (i, 0)),
        ],
        out_specs=pl.BlockSpec((BLK, D), lambda i: (i, 0)),
        out_shape=jax.ShapeDtypeStruct((N, D), jnp.float32),
    )(pout, g, avec, cvec)


def _out_body(g_ref, dinv_ref, o_ref):
    h = g_ref[...] / dinv_ref[...]
    m = jnp.max(h, axis=1, keepdims=True)
    ex = jnp.exp(h - m)
    lse = jnp.log(jnp.sum(ex, axis=1, keepdims=True))
    o_ref[...] = h - m - lse


def _out_stage(g, dinv):
    grid = (N // BLK,)
    return pl.pallas_call(
        _out_body,
        grid=grid,
        in_specs=[
            pl.BlockSpec((BLK, D), lambda i: (i, 0)),
            pl.BlockSpec((BLK, 1), lambda i: (i, 0)),
        ],
        out_specs=pl.BlockSpec((BLK, D), lambda i: (i, 0)),
        out_shape=jax.ShapeDtypeStruct((N, D), jnp.float32),
    )(g, dinv)


def kernel(x, edge_index, W1, b1, W2, b2):
    row = edge_index[0].astype(jnp.int32)
    col = edge_index[1].astype(jnp.int32)
    idx3d = jnp.concatenate(
        [row.reshape(NSU, SUK, U), col.reshape(NSU, SUK, U)], axis=1)
    deg2 = _deg_kernel(idx3d)
    g, avec, cvec, dinv = _mlp_stage(x, deg2, W1, b1, W2, b2)
    for _ in range(K):
        pout = _scatter_kernel(g, idx3d)
        g = _combine_stage(pout, g, avec, cvec)
    return _out_stage(g, dinv)
